# Initial kernel scaffold; baseline (speedup 1.0000x reference)
import jax
import jax.numpy as jnp
from jax.experimental import pallas as pl


def kernel(z, embedding):
    raise NotImplementedError("write your pallas kernel here")


# speedup vs baseline: 2.7299x; 2.7299x over previous
import jax
import jax.numpy as jnp
from jax.experimental import pallas as pl

N_E = 1024
E_DIM = 256
HW = 1024
B = 8


def _vq_body(z_ref, e_ref, zq_ref, enc_ref, idx_ref):
    zb = z_ref[0]
    emb = e_ref[...]
    z_sq = jnp.sum(zb * zb, axis=0, keepdims=True)
    e_sq = jnp.sum(emb * emb, axis=1, keepdims=True)
    mm = jnp.dot(emb, zb, preferred_element_type=jnp.float32)
    scores = (z_sq + e_sq) - 2.0 * mm
    m = jnp.min(scores, axis=0, keepdims=True)
    row_iota = jax.lax.broadcasted_iota(jnp.int32, scores.shape, 0)
    idx = jnp.min(jnp.where(scores == m, row_iota, N_E), axis=0)
    idx_ref[0, 0] = idx
    col_iota = jax.lax.broadcasted_iota(jnp.int32, (HW, N_E), 1)
    onehot = (col_iota == idx[:, None]).astype(jnp.float32)
    enc_ref[...] = onehot
    zq_ref[0] = jax.lax.dot_general(
        emb, onehot, (((0,), (1,)), ((), ())),
        preferred_element_type=jnp.float32)


@jax.jit
def kernel(z, embedding):
    z3 = z.reshape(B, E_DIM, HW)
    zq, enc, idx = pl.pallas_call(
        _vq_body,
        grid=(B,),
        in_specs=[
            pl.BlockSpec((1, E_DIM, HW), lambda b: (b, 0, 0)),
            pl.BlockSpec((N_E, E_DIM), lambda b: (0, 0)),
        ],
        out_specs=[
            pl.BlockSpec((1, E_DIM, HW), lambda b: (b, 0, 0)),
            pl.BlockSpec((HW, N_E), lambda b: (b, 0)),
            pl.BlockSpec((1, 1, HW), lambda b: (b, 0, 0)),
        ],
        out_shape=[
            jax.ShapeDtypeStruct((B, E_DIM, HW), jnp.float32),
            jax.ShapeDtypeStruct((B * HW, N_E), jnp.float32),
            jax.ShapeDtypeStruct((B, 1, HW), jnp.int32),
        ],
    )(z3, embedding)
    z_q = zq.reshape(B, E_DIM, 32, 32)
    return (z_q, (enc, idx.reshape(B * HW, 1)))
